# Initial kernel scaffold; baseline (speedup 1.0000x reference)
import jax
import jax.numpy as jnp
from jax.experimental import pallas as pl


def kernel(x_students, x_assignments, edge_index_s2a, edge_index_a2s, params):
    raise NotImplementedError("write your pallas kernel here")


# speedup vs baseline: 1.8243x; 1.8243x over previous
import functools

import jax
import jax.numpy as jnp
from jax import lax
from jax.experimental import pallas as pl
from jax.experimental.pallas import tpu as pltpu
from jax.experimental.pallas import tpu_sc as plsc

N = 50000
D = 128
E = 500000
CK = 8
HCH = D // CK
ACC = 51200
STRIPE = ACC // 16
TRASH = N
E_PAD = 524288
ROWS2D = E_PAD // 128
BN = 1000
GRID = N // BN
EPS = 1e-5

@functools.cache
def _make_segsum_sc():
    mesh = plsc.VectorSubcoreMesh(core_axis_name="c", subcore_axis_name="s")
    return pl.kernel(
        _segsum_body,
        out_type=jax.ShapeDtypeStruct((CK * ACC, HCH), jnp.float32),
        mesh=mesh,
        scratch_types=[
            pltpu.VMEM((8, 128), jnp.int32),
            pltpu.VMEM((8, 128), jnp.int32),
            pltpu.VMEM((8, 128, HCH), jnp.float32),
            pltpu.VMEM_SHARED((ACC, HCH), jnp.float32),
            pltpu.SemaphoreType.DMA,
        ],
        compiler_params=pltpu.CompilerParams(use_tc_tiling_on_sc=False),
    )


def _segsum_body(xf, src_all, dst2d, zeros_hbm, out, src_v, dst_v, rows_v,
                 acc, sem):
    c = lax.axis_index("c")
    s = lax.axis_index("s")
    for i in range(CK // 2):
        chunk = (CK // 2) * c + i

        pltpu.sync_copy(zeros_hbm.at[pl.ds(s * STRIPE, STRIPE)],
                        acc.at[pl.ds(s * STRIPE, STRIPE)])
        plsc.subcore_barrier()

        row0 = s * 256

        def batch_body(b, carry):
            r = row0 + b * 8
            pltpu.sync_copy(src_all.at[chunk, pl.ds(r, 8)], src_v)
            pltpu.sync_copy(dst2d.at[pl.ds(r, 8)], dst_v)
            descs = []
            for j in range(8):
                descs.append(
                    pltpu.async_copy(xf.at[src_v.at[j]], rows_v.at[j], sem))
            for dsc in descs:
                dsc.wait()
            for j in range(8):
                pltpu.sync_copy(rows_v.at[j], acc.at[dst_v.at[j]], add=True)
            return carry

        lax.fori_loop(0, 32, batch_body, 0)
        plsc.subcore_barrier()
        pltpu.sync_copy(
            acc.at[pl.ds(s * STRIPE, STRIPE)],
            out.at[pl.ds(chunk * ACC + s * STRIPE, STRIPE)])
        plsc.subcore_barrier()


@functools.cache
def _make_counts_sc():
    mesh = plsc.VectorSubcoreMesh(core_axis_name="c", subcore_axis_name="s")
    return pl.kernel(
        _counts_body,
        out_type=jax.ShapeDtypeStruct((2 * ACC, 16), jnp.float32),
        mesh=mesh,
        scratch_types=[
            pltpu.VMEM((8, 128), jnp.int32),
            pltpu.VMEM((128, 16), jnp.float32),
            pltpu.VMEM_SHARED((ACC, 16), jnp.float32),
        ],
        compiler_params=pltpu.CompilerParams(use_tc_tiling_on_sc=False),
    )


def _counts_body(dst_both, ones_hbm, zeros_hbm, out, dst_v, ones_v, cnt):
    c = lax.axis_index("c")
    s = lax.axis_index("s")
    pltpu.sync_copy(ones_hbm, ones_v)
    pltpu.sync_copy(zeros_hbm.at[pl.ds(s * STRIPE, STRIPE)],
                    cnt.at[pl.ds(s * STRIPE, STRIPE)])
    plsc.subcore_barrier()

    row0 = s * 256

    def batch_body(b, carry):
        r = row0 + b * 8
        pltpu.sync_copy(dst_both.at[c, pl.ds(r, 8)], dst_v)
        for j in range(8):
            pltpu.sync_copy(ones_v, cnt.at[dst_v.at[j]], add=True)
        return carry

    lax.fori_loop(0, 32, batch_body, 0)
    plsc.subcore_barrier()
    pltpu.sync_copy(
        cnt.at[pl.ds(s * STRIPE, STRIPE)],
        out.at[pl.ds(c * ACC + s * STRIPE, STRIPE)])
    plsc.subcore_barrier()


def _in_proj_body(x_ref, w_ref, b_ref, o_ref):
    h = jnp.dot(x_ref[...], w_ref[...], preferred_element_type=jnp.float32)
    h = jnp.maximum(h + b_ref[...], 0.0)
    for ch in range(CK):
        o_ref[ch] = h[:, HCH * ch:HCH * (ch + 1)]


def _in_proj(x, w, b):
    return pl.pallas_call(
        _in_proj_body,
        grid=(GRID,),
        in_specs=[
            pl.BlockSpec((BN, D), lambda i: (i, 0)),
            pl.BlockSpec((D, D), lambda i: (0, 0)),
            pl.BlockSpec((1, D), lambda i: (0, 0)),
        ],
        out_specs=pl.BlockSpec((CK, BN, HCH), lambda i: (0, i, 0)),
        out_shape=jax.ShapeDtypeStruct((CK, N, HCH), jnp.float32),
    )(x, w, b.reshape(1, D))


def _post_math(sum_ref, cnt_ref, xd_ref, wl_ref, bl_ref, wr_ref, g_ref, be_ref):
    cnt = cnt_ref[...][:, 0:1]
    recip = 1.0 / jnp.maximum(cnt, 1.0)
    h = jnp.zeros((BN, D), jnp.float32)
    for ch in range(CK):
        m = sum_ref[ch] * recip
        h = h + jnp.dot(m, wl_ref[HCH * ch:HCH * (ch + 1), :],
                        preferred_element_type=jnp.float32)
        h = h + jnp.dot(xd_ref[ch], wr_ref[HCH * ch:HCH * (ch + 1), :],
                        preferred_element_type=jnp.float32)
    h = h + bl_ref[...]
    mu = jnp.mean(h, axis=1, keepdims=True)
    var = jnp.mean((h - mu) * (h - mu), axis=1, keepdims=True)
    ln = (h - mu) * lax.rsqrt(var + EPS) * g_ref[...] + be_ref[...]
    return jnp.maximum(ln, 0.0)


def _sage_post_body(sum_ref, cnt_ref, xd_ref, wl_ref, bl_ref, wr_ref, g_ref,
                    be_ref, o_ref):
    o = _post_math(sum_ref, cnt_ref, xd_ref, wl_ref, bl_ref, wr_ref, g_ref,
                   be_ref)
    for ch in range(CK):
        o_ref[ch] = o[:, HCH * ch:HCH * (ch + 1)]


def _sage_final_body(sum_ref, cnt_ref, xd_ref, wl_ref, bl_ref, wr_ref, g_ref,
                     be_ref, wo_ref, bo_ref, o_ref):
    o = _post_math(sum_ref, cnt_ref, xd_ref, wl_ref, bl_ref, wr_ref, g_ref,
                   be_ref)
    o_ref[...] = (jnp.dot(o, wo_ref[...], preferred_element_type=jnp.float32)
                  + bo_ref[...])


_COMMON_SPECS = [
    pl.BlockSpec((CK, BN, HCH), lambda i: (0, i, 0)),
    pl.BlockSpec((BN, 16), lambda i: (i, 0)),
    pl.BlockSpec((CK, BN, HCH), lambda i: (0, i, 0)),
    pl.BlockSpec((D, D), lambda i: (0, 0)),
    pl.BlockSpec((1, D), lambda i: (0, 0)),
    pl.BlockSpec((D, D), lambda i: (0, 0)),
    pl.BlockSpec((1, D), lambda i: (0, 0)),
    pl.BlockSpec((1, D), lambda i: (0, 0)),
]


def _sage_post(summed, cnt, xd, wl, bl, wr, g, be):
    return pl.pallas_call(
        _sage_post_body,
        grid=(GRID,),
        in_specs=_COMMON_SPECS,
        out_specs=pl.BlockSpec((CK, BN, HCH), lambda i: (0, i, 0)),
        out_shape=jax.ShapeDtypeStruct((CK, N, HCH), jnp.float32),
    )(summed, cnt, xd, wl, bl.reshape(1, D), wr, g.reshape(1, D),
      be.reshape(1, D))


def _sage_final(summed, cnt, xd, wl, bl, wr, g, be, wo, bo, base):
    return pl.pallas_call(
        _sage_final_body,
        grid=(GRID,),
        in_specs=_COMMON_SPECS + [
            pl.BlockSpec((D, 1), lambda i: (0, 0)),
            pl.BlockSpec((1, 1), lambda i: (0, 0)),
        ],
        out_specs=pl.BlockSpec((BN, 1), lambda i: (i, 0)),
        out_shape=jax.ShapeDtypeStruct((N, 1), jnp.float32),
    )(summed, cnt, xd, wl, bl.reshape(1, D), wr, g.reshape(1, D),
      be.reshape(1, D), wo, (bo + base).reshape(1, 1))


def _pad_idx(idx, fill):
    pad = jnp.full((E_PAD - E,), fill, dtype=jnp.int32)
    return jnp.concatenate([idx.astype(jnp.int32), pad]).reshape(ROWS2D, 128)


def kernel(x_students, x_assignments, edge_index_s2a, edge_index_a2s, params):
    p = params
    chunk_off = (jnp.arange(CK, dtype=jnp.int32) * N)[:, None, None]
    src_s2a = _pad_idx(edge_index_s2a[0], 0)[None] + chunk_off
    dst_s2a = _pad_idx(edge_index_s2a[1], TRASH)
    src_a2s = _pad_idx(edge_index_a2s[0], 0)[None] + chunk_off
    dst_a2s = _pad_idx(edge_index_a2s[1], TRASH)
    dst_both = jnp.stack([dst_s2a, dst_a2s])

    zeros_acc = jnp.zeros((ACC, HCH), jnp.float32)
    ones16 = jnp.ones((128, 16), jnp.float32)
    zeros16 = jnp.zeros((ACC, 16), jnp.float32)

    xs0 = _in_proj(x_students, p['W_in_s'], p['b_in_s'])
    xa0 = _in_proj(x_assignments, p['W_in_a'], p['b_in_a'])

    cnt = _make_counts_sc()(dst_both, ones16, zeros16).reshape(2, ACC, 16)
    cnt_a, cnt_s = cnt[0], cnt[1]

    def seg(x_ch, src2d, dst2d):
        flat = x_ch.reshape(CK * N, HCH)
        return _make_segsum_sc()(flat, src2d, dst2d,
                                 zeros_acc).reshape(CK, ACC, HCH)

    sum_a0 = seg(xs0, src_s2a, dst_s2a)
    xa1 = _sage_post(sum_a0, cnt_a, xa0, p['Wl_s2a'][0], p['bl_s2a'][0],
                     p['Wr_s2a'][0], p['g_a'][0], p['be_a'][0])

    sum_s0 = seg(xa0, src_a2s, dst_a2s)
    xs1 = _sage_post(sum_s0, cnt_s, xs0, p['Wl_a2s'][0], p['bl_a2s'][0],
                     p['Wr_a2s'][0], p['g_s'][0], p['be_s'][0])

    sum_a1 = seg(xs1, src_s2a, dst_s2a)
    out = _sage_final(sum_a1, cnt_a, xa1, p['Wl_s2a'][1], p['bl_s2a'][1],
                      p['Wr_s2a'][1], p['g_a'][1], p['be_a'][1],
                      p['W_out'], p['b_out'], p['base'])
    return out[:, 0]


# speedup vs baseline: 1.8480x; 1.0130x over previous
import functools

import jax
import jax.numpy as jnp
from jax import lax
from jax.experimental import pallas as pl
from jax.experimental.pallas import tpu as pltpu
from jax.experimental.pallas import tpu_sc as plsc

N = 50000
D = 128
E = 500000
CK = 8
HCH = D // CK
ACC = 51200
STRIPE = ACC // 16
TRASH = N
E_PAD = 524288
ROWS2D = E_PAD // 128
BN = 1000
GRID = N // BN
EPS = 1e-5

@functools.cache
def _make_segsum_sc():
    mesh = plsc.VectorSubcoreMesh(core_axis_name="c", subcore_axis_name="s")
    return pl.kernel(
        _segsum_body,
        out_type=jax.ShapeDtypeStruct((CK * ACC, HCH), jnp.float32),
        mesh=mesh,
        scratch_types=[
            pltpu.VMEM((8, 128), jnp.int32),
            pltpu.VMEM((8, 128), jnp.int32),
            pltpu.VMEM((8, 128, HCH), jnp.float32),
            pltpu.VMEM_SHARED((ACC, HCH), jnp.float32),
            pltpu.SemaphoreType.DMA,
            pltpu.SemaphoreType.DMA,
        ],
        compiler_params=pltpu.CompilerParams(use_tc_tiling_on_sc=False),
    )


def _segsum_body(xf, src_all, dst2d, zeros_hbm, out, src_v, dst_v, rows_v,
                 acc, sem, sem2):
    c = lax.axis_index("c")
    s = lax.axis_index("s")
    for i in range(CK // 2):
        chunk = (CK // 2) * c + i

        pltpu.sync_copy(zeros_hbm.at[pl.ds(s * STRIPE, STRIPE)],
                        acc.at[pl.ds(s * STRIPE, STRIPE)])
        plsc.subcore_barrier()

        row0 = s * 256

        def batch_body(b, carry):
            r = row0 + b * 8
            pltpu.sync_copy(src_all.at[chunk, pl.ds(r, 8)], src_v)
            pltpu.sync_copy(dst2d.at[pl.ds(r, 8)], dst_v)
            descs = []
            for j in range(8):
                descs.append(
                    pltpu.async_copy(xf.at[src_v.at[j]], rows_v.at[j], sem))
            for dsc in descs:
                dsc.wait()
            sdescs = []
            for j in range(8):
                sdescs.append(
                    pltpu.async_copy(rows_v.at[j], acc.at[dst_v.at[j]], sem2,
                                     add=True))
            for dsc in sdescs:
                dsc.wait()
            return carry

        lax.fori_loop(0, 32, batch_body, 0)
        plsc.subcore_barrier()
        pltpu.sync_copy(
            acc.at[pl.ds(s * STRIPE, STRIPE)],
            out.at[pl.ds(chunk * ACC + s * STRIPE, STRIPE)])
        plsc.subcore_barrier()


@functools.cache
def _make_counts_sc():
    mesh = plsc.VectorSubcoreMesh(core_axis_name="c", subcore_axis_name="s")
    return pl.kernel(
        _counts_body,
        out_type=jax.ShapeDtypeStruct((2 * ACC, 16), jnp.float32),
        mesh=mesh,
        scratch_types=[
            pltpu.VMEM((8, 128), jnp.int32),
            pltpu.VMEM((128, 16), jnp.float32),
            pltpu.VMEM_SHARED((ACC, 16), jnp.float32),
            pltpu.SemaphoreType.DMA,
        ],
        compiler_params=pltpu.CompilerParams(use_tc_tiling_on_sc=False),
    )


def _counts_body(dst_both, ones_hbm, zeros_hbm, out, dst_v, ones_v, cnt, sem):
    c = lax.axis_index("c")
    s = lax.axis_index("s")
    pltpu.sync_copy(ones_hbm, ones_v)
    pltpu.sync_copy(zeros_hbm.at[pl.ds(s * STRIPE, STRIPE)],
                    cnt.at[pl.ds(s * STRIPE, STRIPE)])
    plsc.subcore_barrier()

    row0 = s * 256

    def batch_body(b, carry):
        r = row0 + b * 8
        pltpu.sync_copy(dst_both.at[c, pl.ds(r, 8)], dst_v)
        sdescs = []
        for j in range(8):
            sdescs.append(
                pltpu.async_copy(ones_v, cnt.at[dst_v.at[j]], sem, add=True))
        for dsc in sdescs:
            dsc.wait()
        return carry

    lax.fori_loop(0, 32, batch_body, 0)
    plsc.subcore_barrier()
    pltpu.sync_copy(
        cnt.at[pl.ds(s * STRIPE, STRIPE)],
        out.at[pl.ds(c * ACC + s * STRIPE, STRIPE)])
    plsc.subcore_barrier()


def _in_proj_body(x_ref, w_ref, b_ref, o_ref):
    h = jnp.dot(x_ref[...], w_ref[...], preferred_element_type=jnp.float32)
    h = jnp.maximum(h + b_ref[...], 0.0)
    for ch in range(CK):
        o_ref[ch] = h[:, HCH * ch:HCH * (ch + 1)]


def _in_proj(x, w, b):
    return pl.pallas_call(
        _in_proj_body,
        grid=(GRID,),
        in_specs=[
            pl.BlockSpec((BN, D), lambda i: (i, 0)),
            pl.BlockSpec((D, D), lambda i: (0, 0)),
            pl.BlockSpec((1, D), lambda i: (0, 0)),
        ],
        out_specs=pl.BlockSpec((CK, BN, HCH), lambda i: (0, i, 0)),
        out_shape=jax.ShapeDtypeStruct((CK, N, HCH), jnp.float32),
    )(x, w, b.reshape(1, D))


def _post_math(sum_ref, cnt_ref, xd_ref, wl_ref, bl_ref, wr_ref, g_ref, be_ref):
    cnt = cnt_ref[...][:, 0:1]
    recip = 1.0 / jnp.maximum(cnt, 1.0)
    h = jnp.zeros((BN, D), jnp.float32)
    for ch in range(CK):
        m = sum_ref[ch] * recip
        h = h + jnp.dot(m, wl_ref[HCH * ch:HCH * (ch + 1), :],
                        preferred_element_type=jnp.float32)
        h = h + jnp.dot(xd_ref[ch], wr_ref[HCH * ch:HCH * (ch + 1), :],
                        preferred_element_type=jnp.float32)
    h = h + bl_ref[...]
    mu = jnp.mean(h, axis=1, keepdims=True)
    var = jnp.mean((h - mu) * (h - mu), axis=1, keepdims=True)
    ln = (h - mu) * lax.rsqrt(var + EPS) * g_ref[...] + be_ref[...]
    return jnp.maximum(ln, 0.0)


def _sage_post_body(sum_ref, cnt_ref, xd_ref, wl_ref, bl_ref, wr_ref, g_ref,
                    be_ref, o_ref):
    o = _post_math(sum_ref, cnt_ref, xd_ref, wl_ref, bl_ref, wr_ref, g_ref,
                   be_ref)
    for ch in range(CK):
        o_ref[ch] = o[:, HCH * ch:HCH * (ch + 1)]


def _sage_final_body(sum_ref, cnt_ref, xd_ref, wl_ref, bl_ref, wr_ref, g_ref,
                     be_ref, wo_ref, bo_ref, o_ref):
    o = _post_math(sum_ref, cnt_ref, xd_ref, wl_ref, bl_ref, wr_ref, g_ref,
                   be_ref)
    o_ref[...] = (jnp.dot(o, wo_ref[...], preferred_element_type=jnp.float32)
                  + bo_ref[...])


_COMMON_SPECS = [
    pl.BlockSpec((CK, BN, HCH), lambda i: (0, i, 0)),
    pl.BlockSpec((BN, 16), lambda i: (i, 0)),
    pl.BlockSpec((CK, BN, HCH), lambda i: (0, i, 0)),
    pl.BlockSpec((D, D), lambda i: (0, 0)),
    pl.BlockSpec((1, D), lambda i: (0, 0)),
    pl.BlockSpec((D, D), lambda i: (0, 0)),
    pl.BlockSpec((1, D), lambda i: (0, 0)),
    pl.BlockSpec((1, D), lambda i: (0, 0)),
]


def _sage_post(summed, cnt, xd, wl, bl, wr, g, be):
    return pl.pallas_call(
        _sage_post_body,
        grid=(GRID,),
        in_specs=_COMMON_SPECS,
        out_specs=pl.BlockSpec((CK, BN, HCH), lambda i: (0, i, 0)),
        out_shape=jax.ShapeDtypeStruct((CK, N, HCH), jnp.float32),
    )(summed, cnt, xd, wl, bl.reshape(1, D), wr, g.reshape(1, D),
      be.reshape(1, D))


def _sage_final(summed, cnt, xd, wl, bl, wr, g, be, wo, bo, base):
    return pl.pallas_call(
        _sage_final_body,
        grid=(GRID,),
        in_specs=_COMMON_SPECS + [
            pl.BlockSpec((D, 1), lambda i: (0, 0)),
            pl.BlockSpec((1, 1), lambda i: (0, 0)),
        ],
        out_specs=pl.BlockSpec((BN, 1), lambda i: (i, 0)),
        out_shape=jax.ShapeDtypeStruct((N, 1), jnp.float32),
    )(summed, cnt, xd, wl, bl.reshape(1, D), wr, g.reshape(1, D),
      be.reshape(1, D), wo, (bo + base).reshape(1, 1))


def _pad_idx(idx, fill):
    pad = jnp.full((E_PAD - E,), fill, dtype=jnp.int32)
    return jnp.concatenate([idx.astype(jnp.int32), pad]).reshape(ROWS2D, 128)


def kernel(x_students, x_assignments, edge_index_s2a, edge_index_a2s, params):
    p = params
    chunk_off = (jnp.arange(CK, dtype=jnp.int32) * N)[:, None, None]
    src_s2a = _pad_idx(edge_index_s2a[0], 0)[None] + chunk_off
    dst_s2a = _pad_idx(edge_index_s2a[1], TRASH)
    src_a2s = _pad_idx(edge_index_a2s[0], 0)[None] + chunk_off
    dst_a2s = _pad_idx(edge_index_a2s[1], TRASH)
    dst_both = jnp.stack([dst_s2a, dst_a2s])

    zeros_acc = jnp.zeros((ACC, HCH), jnp.float32)
    ones16 = jnp.ones((128, 16), jnp.float32)
    zeros16 = jnp.zeros((ACC, 16), jnp.float32)

    xs0 = _in_proj(x_students, p['W_in_s'], p['b_in_s'])
    xa0 = _in_proj(x_assignments, p['W_in_a'], p['b_in_a'])

    cnt = _make_counts_sc()(dst_both, ones16, zeros16).reshape(2, ACC, 16)
    cnt_a, cnt_s = cnt[0], cnt[1]

    def seg(x_ch, src2d, dst2d):
        flat = x_ch.reshape(CK * N, HCH)
        return _make_segsum_sc()(flat, src2d, dst2d,
                                 zeros_acc).reshape(CK, ACC, HCH)

    sum_a0 = seg(xs0, src_s2a, dst_s2a)
    xa1 = _sage_post(sum_a0, cnt_a, xa0, p['Wl_s2a'][0], p['bl_s2a'][0],
                     p['Wr_s2a'][0], p['g_a'][0], p['be_a'][0])

    sum_s0 = seg(xa0, src_a2s, dst_a2s)
    xs1 = _sage_post(sum_s0, cnt_s, xs0, p['Wl_a2s'][0], p['bl_a2s'][0],
                     p['Wr_a2s'][0], p['g_s'][0], p['be_s'][0])

    sum_a1 = seg(xs1, src_s2a, dst_s2a)
    out = _sage_final(sum_a1, cnt_a, xa1, p['Wl_s2a'][1], p['bl_s2a'][1],
                      p['Wr_s2a'][1], p['g_a'][1], p['be_a'][1],
                      p['W_out'], p['b_out'], p['base'])
    return out[:, 0]
